# Initial kernel scaffold; baseline (speedup 1.0000x reference)
import jax
import jax.numpy as jnp
from jax.experimental import pallas as pl


def kernel(x, drug_indices, W_shared, b_shared, W_pw, b_pw, W_drug, b_drug):
    raise NotImplementedError("write your pallas kernel here")


# speedup vs baseline: 10.9965x; 10.9965x over previous
import jax
import jax.numpy as jnp
from jax.experimental import pallas as pl

_BATCH = 4096
_IN = 2048
_SH = 256
_PW = 128
_NP = 16
_ND = 64
_BB = 512


def _tc_body(x_ref, drug_ref, ws_ref, bs_ref, wp_ref, bp_ref, wdr_ref,
             bdr_ref, o_ref):
    xb = x_ref[...]
    h = jnp.maximum(
        jnp.dot(xb, ws_ref[...], preferred_element_type=jnp.float32)
        + bs_ref[...], 0.0)
    z = jnp.dot(h, wp_ref[...], preferred_element_type=jnp.float32) + bp_ref[...]
    a = jnp.maximum(z, 0.0)
    drug = drug_ref[...]
    oh = (drug == jax.lax.broadcasted_iota(jnp.int32, (_BB, _ND), 1)
          ).astype(jnp.float32)
    wd = jnp.dot(oh, wdr_ref[...], preferred_element_type=jnp.float32)
    bd = jnp.dot(oh, bdr_ref[...], preferred_element_type=jnp.float32)
    pw = drug % _NP
    colp = jax.lax.broadcasted_iota(jnp.int32, (_BB, _NP * _PW), 1) // _PW
    wd_t = jnp.concatenate([wd] * _NP, axis=1)
    mw = jnp.where(colp == pw, wd_t, 0.0)
    o_ref[...] = jnp.sum(a * mw, axis=1, keepdims=True) + bd


def kernel(x, drug_indices, W_shared, b_shared, W_pw, b_pw, W_drug, b_drug):
    wp_flat = jnp.transpose(W_pw, (1, 0, 2)).reshape(_SH, _NP * _PW)
    bp_flat = b_pw.reshape(1, _NP * _PW)
    drug2d = drug_indices.reshape(_BATCH, 1)
    bs2d = b_shared.reshape(1, _SH)
    bd2d = b_drug.reshape(_ND, 1)

    grid = (_BATCH // _BB,)
    out = pl.pallas_call(
        _tc_body,
        grid=grid,
        in_specs=[
            pl.BlockSpec((_BB, _IN), lambda i: (i, 0)),
            pl.BlockSpec((_BB, 1), lambda i: (i, 0)),
            pl.BlockSpec((_IN, _SH), lambda i: (0, 0)),
            pl.BlockSpec((1, _SH), lambda i: (0, 0)),
            pl.BlockSpec((_SH, _NP * _PW), lambda i: (0, 0)),
            pl.BlockSpec((1, _NP * _PW), lambda i: (0, 0)),
            pl.BlockSpec((_ND, _PW), lambda i: (0, 0)),
            pl.BlockSpec((_ND, 1), lambda i: (0, 0)),
        ],
        out_specs=pl.BlockSpec((_BB, 1), lambda i: (i, 0)),
        out_shape=jax.ShapeDtypeStruct((_BATCH, 1), jnp.float32),
    )(x, drug2d, W_shared, bs2d, wp_flat, bp_flat, W_drug, bd2d)
    return out.reshape(_BATCH)
